# baseline (speedup 1.0000x reference)
import functools

import jax
import jax.numpy as jnp
from jax import lax
from jax.experimental import pallas as pl
from jax.experimental.pallas import tpu as pltpu
from jax.experimental.pallas import tpu_sc as plsc

H = 128
NC = 2
NS = 16
EB = 40
CMAX = 12544
DUMMY = CMAX


def _mlp_body(x_ref, w1_ref, b1_ref, w2_ref, b2_ref, w3_ref, b3_ref, o_ref):
    x = x_ref[...]
    h = jnp.maximum(jnp.dot(x, w1_ref[...], preferred_element_type=jnp.float32) + b1_ref[...], 0.0)
    h = jnp.maximum(jnp.dot(h, w2_ref[...], preferred_element_type=jnp.float32) + b2_ref[...], 0.0)
    o_ref[...] = jnp.dot(h, w3_ref[...], preferred_element_type=jnp.float32) + b3_ref[...]


def _mlp(x, p, name, blk=1000):
    n = x.shape[0]
    d = x.shape[1]
    w1 = p["mlp_%s_W1" % name]; b1 = p["mlp_%s_b1" % name].reshape(1, H)
    w2 = p["mlp_%s_W2" % name]; b2 = p["mlp_%s_b2" % name].reshape(1, H)
    w3 = p["mlp_%s_W3" % name]; b3 = p["mlp_%s_b3" % name].reshape(1, H)
    full = lambda r, c: pl.BlockSpec((r, c), lambda i: (0, 0))
    return pl.pallas_call(
        _mlp_body,
        grid=(n // blk,),
        in_specs=[
            pl.BlockSpec((blk, d), lambda i: (i, 0)),
            full(d, H), full(1, H), full(H, H), full(1, H), full(H, H), full(1, H),
        ],
        out_specs=pl.BlockSpec((blk, H), lambda i: (i, 0)),
        out_shape=jax.ShapeDtypeStruct((n, H), jnp.float32),
    )(x, w1, b1, w2, b2, w3, b3)


def _combine2_body(s1_ref, c1_ref, s2_ref, c2_ref, x_ref, wl1_ref, wl2_ref,
                   wr1_ref, wr2_ref, b_ref, o_ref):
    inv1 = 1.0 / jnp.maximum(c1_ref[:, 0:1], 1.0)
    inv2 = 1.0 / jnp.maximum(c2_ref[:, 0:1], 1.0)
    m1 = s1_ref[...] * inv1
    m2 = s2_ref[...] * inv2
    acc = jnp.dot(m1, wl1_ref[...], preferred_element_type=jnp.float32)
    acc += jnp.dot(m2, wl2_ref[...], preferred_element_type=jnp.float32)
    acc += jnp.dot(x_ref[...], wr1_ref[...] + wr2_ref[...], preferred_element_type=jnp.float32)
    o_ref[...] = jnp.maximum(acc + b_ref[...], 0.0)


def _combine2(s1, c1, s2, c2, x, wl1, wl2, wr1, wr2, b, blk=1000):
    n = x.shape[0]
    full = lambda r, c: pl.BlockSpec((r, c), lambda i: (0, 0))
    row = lambda c: pl.BlockSpec((blk, c), lambda i: (i, 0))
    return pl.pallas_call(
        _combine2_body,
        grid=(n // blk,),
        in_specs=[row(H), row(H), row(H), row(H), row(H),
                  full(H, H), full(H, H), full(H, H), full(H, H), full(1, H)],
        out_specs=row(H),
        out_shape=jax.ShapeDtypeStruct((n, H), jnp.float32),
    )(s1, c1, s2, c2, x, wl1, wl2, wr1, wr2, b)


def _combine1_body(s_ref, c_ref, x_ref, wl_ref, wr_ref, b_ref, o_ref):
    inv = 1.0 / jnp.maximum(c_ref[:, 0:1], 1.0)
    acc = jnp.dot(s_ref[...] * inv, wl_ref[...], preferred_element_type=jnp.float32)
    acc += jnp.dot(x_ref[...], wr_ref[...], preferred_element_type=jnp.float32)
    o_ref[...] = jnp.maximum(acc + b_ref[...], 0.0)


def _combine1(s, c, x, wl, wr, b, blk=1000):
    n = x.shape[0]
    full = lambda r, cc: pl.BlockSpec((r, cc), lambda i: (0, 0))
    row = lambda cc: pl.BlockSpec((blk, cc), lambda i: (i, 0))
    return pl.pallas_call(
        _combine1_body,
        grid=(n // blk,),
        in_specs=[row(H), row(H), row(H), full(H, H), full(H, H), full(1, H)],
        out_specs=row(H),
        out_shape=jax.ShapeDtypeStruct((n, H), jnp.float32),
    )(s, c, x, wl, wr, b)


def _final_body(s_ref, c_ref, x_ref, wl_ref, wr_ref, b_ref, lw_ref, lb_ref,
                bs_ref, o_ref, *, blk):
    inv = 1.0 / jnp.maximum(c_ref[:, 0:1], 1.0)
    acc = jnp.dot(s_ref[...] * inv, wl_ref[...], preferred_element_type=jnp.float32)
    acc += jnp.dot(x_ref[...], wr_ref[...], preferred_element_type=jnp.float32)
    h = jnp.maximum(acc + b_ref[...], 0.0)
    o = jnp.maximum(jnp.dot(h, lw_ref[...], preferred_element_type=jnp.float32) + lb_ref[...], 0.0)
    rows = pl.program_id(0) * blk + lax.broadcasted_iota(jnp.int32, (blk, 1), 0)
    o_ref[...] = jnp.where(rows < bs_ref[0, 0], o, 0.0)


def _final(s, c, x, wl, wr, b, lw, lb, bs, blk=1000):
    n = x.shape[0]
    full = lambda r, cc: pl.BlockSpec((r, cc), lambda i: (0, 0))
    row = lambda cc: pl.BlockSpec((blk, cc), lambda i: (i, 0))
    return pl.pallas_call(
        functools.partial(_final_body, blk=blk),
        grid=(n // blk,),
        in_specs=[row(H), row(H), row(H), full(H, H), full(H, H), full(1, H),
                  full(H, 1), full(1, 1),
                  pl.BlockSpec(memory_space=pltpu.SMEM)],
        out_specs=row(1),
        out_shape=jax.ShapeDtypeStruct((n, 1), jnp.float32),
    )(s, c, x, wl, wr, b, lw, lb, bs)


def _epad(E):
    m = 5120
    return ((E + m - 1) // m) * m


def _sc_chunk_lidx(cols, E, chunk, n_chunks):
    ep = _epad(E)
    er = ep // 128
    br = next(b for b in (512, 320, 256, 160, 128, 64, 32, 16, 8) if er % b == 0)
    cols_p = jnp.pad(cols, (0, ep - E), constant_values=2 ** 30).reshape(er, 128)

    def body(c_ref, o_ref):
        ch = pl.program_id(1)
        lo = ch * chunk
        v = c_ref[...]
        m = (v >= lo) & (v < lo + chunk)
        o_ref[...] = jnp.where(m, v - lo, DUMMY)

    out = pl.pallas_call(
        body,
        grid=(er // br, n_chunks),
        in_specs=[pl.BlockSpec((br, 128), lambda i, ch: (i, 0))],
        out_specs=pl.BlockSpec((br, 128),
                               lambda i, ch, _g=er // br: (ch * _g + i, 0)),
        out_shape=jax.ShapeDtypeStruct((n_chunks * er, 128), jnp.int32),
    )(cols_p)
    return out.reshape(n_chunks * ep)


def _make_sc_segsum(task_shapes):
    out_type = []
    for (E, n_dst, chunk, with_count) in task_shapes:
        assert n_dst % chunk == 0 and chunk % (16 * NS) == 0
        assert (n_dst // chunk) % NC == 0 and chunk <= CMAX
        out_type.append(jax.ShapeDtypeStruct((n_dst, H), jnp.float32))
        if with_count:
            out_type.append(jax.ShapeDtypeStruct((n_dst, H), jnp.float32))

    mesh = plsc.VectorSubcoreMesh(core_axis_name="c", subcore_axis_name="s")
    scratch_types = [
        pltpu.VMEM_SHARED((CMAX + 8, H), jnp.float32),
        pltpu.VMEM((8, H), jnp.float32),
        pltpu.VMEM((EB, H), jnp.float32),
        pltpu.VMEM((EB,), jnp.int32), pltpu.VMEM((EB,), jnp.int32),
        pltpu.VMEM((EB,), jnp.int32), pltpu.VMEM((EB,), jnp.int32),
        pltpu.VMEM((EB, H), jnp.float32), pltpu.VMEM((EB, H), jnp.float32),
        pltpu.SemaphoreType.DMA, pltpu.SemaphoreType.DMA,
    ]

    @functools.partial(pl.kernel, mesh=mesh, out_type=tuple(out_type),
                       scratch_types=scratch_types)
    def sc_kernel(*refs):
        n_task = len(task_shapes)
        ins = refs[:3 * n_task + 1]
        ones_in = ins[3 * n_task]
        outs = list(refs[3 * n_task + 1:len(refs) - len(scratch_types)])
        (feat_acc, zfeat, obuf, rb0, rb1, sb0, sb1, gb0, gb1,
         sem0, sem1) = refs[len(refs) - len(scratch_types):]
        rb = (rb0, rb1); sb = (sb0, sb1); gb = (gb0, gb1); sem = (sem0, sem1)

        cid = lax.axis_index("c")
        sid = lax.axis_index("s")

        zero = jnp.zeros((16,), jnp.float32)
        for r in range(8):
            for k in range(H // 16):
                zfeat[r, pl.ds(16 * k, 16)] = zero
        pltpu.sync_copy(ones_in, obuf)

        oi = 0
        for t, (E, n_dst, chunk, with_count) in enumerate(task_shapes):
            rows_hbm, lidx_hbm, src_hbm = ins[3 * t], ins[3 * t + 1], ins[3 * t + 2]
            s_hbm = outs[oi]; oi += 1
            c_hbm = outs[oi] if with_count else None
            if with_count:
                oi += 1

            ep = _epad(E)
            nsub = ep // EB // NS
            rps = chunk // NS
            my_base = sid * nsub * EB

            def zero_acc(_rps=rps):
                def zbody(i, _b=sid * _rps):
                    pltpu.sync_copy(zfeat, feat_acc.at[pl.ds(_b + i * 8, 8)])
                pl.loop(0, _rps // 8)(zbody)

            def copy_out(dst_hbm, lo, _rps=rps):
                def cpbody(i, _b=sid * _rps, _lo=lo, _d=dst_hbm):
                    r = _b + i * 16
                    pltpu.sync_copy(feat_acc.at[pl.ds(r, 16)],
                                    _d.at[pl.ds(_lo + r, 16)])
                pl.loop(0, _rps // 16)(cpbody)

            for p in range((n_dst // chunk) // NC):
                ch = p * NC + cid
                lo = ch * chunk
                lbase = ch * ep + my_base

                zero_acc()
                plsc.subcore_barrier()

                for b in range(2):
                    off = b * EB
                    pltpu.sync_copy(rows_hbm.at[pl.ds(my_base + off, EB)], rb[b])
                    pltpu.sync_copy(lidx_hbm.at[pl.ds(lbase + off, EB)], sb[b])
                    pltpu.async_copy(src_hbm.at[rb[b]], gb[b], sem[b])

                def fbody(i2, _lb=lbase, _n=nsub):
                    for b in range(2):
                        pltpu.make_async_copy(src_hbm.at[pl.ds(0, EB)], gb[b],
                                              sem[b]).wait()
                        pltpu.sync_copy(gb[b], feat_acc.at[sb[b]], add=True)
                    @pl.when(i2 < _n // 2 - 1)
                    def _():
                        for b in range(2):
                            off = (i2 * 2 + b + 2) * EB
                            pltpu.sync_copy(rows_hbm.at[pl.ds(my_base + off, EB)],
                                            rb[b])
                            pltpu.sync_copy(lidx_hbm.at[pl.ds(_lb + off, EB)],
                                            sb[b])
                            pltpu.async_copy(src_hbm.at[rb[b]], gb[b], sem[b])
                pl.loop(0, nsub // 2)(fbody)
                plsc.subcore_barrier()
                copy_out(s_hbm, lo)

                if with_count:
                    plsc.subcore_barrier()
                    zero_acc()
                    plsc.subcore_barrier()
                    for b in range(2):
                        pltpu.sync_copy(lidx_hbm.at[pl.ds(lbase + b * EB, EB)],
                                        sb[b])
                        pltpu.async_copy(obuf, feat_acc.at[sb[b]], sem[b],
                                         add=True)

                    def cbody(i2, _lb=lbase, _n=nsub):
                        for b in range(2):
                            pltpu.make_async_copy(src_hbm.at[pl.ds(0, EB)],
                                                  gb[b], sem[b]).wait()
                        @pl.when(i2 < _n // 2 - 1)
                        def _():
                            for b in range(2):
                                off = (i2 * 2 + b + 2) * EB
                                pltpu.sync_copy(
                                    lidx_hbm.at[pl.ds(_lb + off, EB)], sb[b])
                                pltpu.async_copy(obuf, feat_acc.at[sb[b]],
                                                 sem[b], add=True)
                    pl.loop(0, nsub // 2)(cbody)
                    plsc.subcore_barrier()
                    copy_out(c_hbm, lo)

    return sc_kernel


N_SNP_PAD = 50176
N_GENE_PAD = 20480


def _pad_rows(ei):
    E = ei.shape[1]
    return jnp.pad(ei[0], (0, _epad(E) - E))


def _sc_layer1(h_snp, h_gene, ei_sg, ei_gg, ei_gs, lidx_gs):
    lidx_sg = _sc_chunk_lidx(ei_sg[1], 200000, 10240, 2)
    lidx_gg = _sc_chunk_lidx(ei_gg[1], 100000, 10240, 2)
    ones = jnp.ones((EB, H), jnp.float32)
    k = _make_sc_segsum([
        (200000, N_GENE_PAD, 10240, True),
        (100000, N_GENE_PAD, 10240, True),
        (200000, N_SNP_PAD, CMAX, True),
    ])
    return k(
        _pad_rows(ei_sg), lidx_sg, h_snp,
        _pad_rows(ei_gg), lidx_gg, h_gene,
        _pad_rows(ei_gs), lidx_gs, h_gene, ones)


def _sc_layer2(gene1, ei_gs, lidx_gs):
    ones = jnp.ones((EB, H), jnp.float32)
    k = _make_sc_segsum([
        (200000, N_SNP_PAD, CMAX, False),
    ])
    return k(_pad_rows(ei_gs), lidx_gs, gene1, ones)


def kernel(x_SNP, x_Gene, x_CC, x_BP, x_MF, ei_snp_gene, ei_gene_snp,
           ei_gene_gene, ei_gene_cc, ei_gene_bp, ei_gene_mf, params, batch_size):
    p = params
    n_snp = x_SNP.shape[0]
    n_gene = x_Gene.shape[0]

    h_snp = _mlp(x_SNP, p, "snp")
    h_gene = _mlp(x_Gene, p, "gene")

    lidx_gs = _sc_chunk_lidx(ei_gene_snp[1], 200000, CMAX, 4)

    s_sg, c_sg, s_gg, c_gg, s_gs_p, c_gs_p = _sc_layer1(
        h_snp, h_gene, ei_snp_gene, ei_gene_gene, ei_gene_snp, lidx_gs)
    s_sg, c_sg = s_sg[:n_gene], c_sg[:n_gene]
    s_gg, c_gg = s_gg[:n_gene], c_gg[:n_gene]
    s_gs = s_gs_p[:n_snp]
    c_gs = c_gs_p[:n_snp]

    gene1 = _combine2(
        s_sg, c_sg, s_gg, c_gg, h_gene,
        p["conv0_snp_gene_Wl"], p["conv0_gene_gene_Wl"],
        p["conv0_snp_gene_Wr"], p["conv0_gene_gene_Wr"],
        (p["conv0_snp_gene_bl"] + p["conv0_gene_gene_bl"]).reshape(1, H))
    snp1 = _combine1(
        s_gs, c_gs, h_snp,
        p["conv0_gene_snp_Wl"], p["conv0_gene_snp_Wr"],
        p["conv0_gene_snp_bl"].reshape(1, H))

    s2 = _sc_layer2(gene1, ei_gene_snp, lidx_gs)
    s2_gs = (s2[0] if isinstance(s2, (tuple, list)) else s2)[:n_snp]

    bs = jnp.asarray(batch_size, jnp.int32).reshape(1, 1)
    out = _final(
        s2_gs, c_gs, snp1,
        p["conv1_gene_snp_Wl"], p["conv1_gene_snp_Wr"],
        p["conv1_gene_snp_bl"].reshape(1, H),
        p["lin_W"], p["lin_b"].reshape(1, 1), bs)
    return out


# speedup vs baseline: 1.1391x; 1.1391x over previous
import functools

import jax
import jax.numpy as jnp
from jax import lax
from jax.experimental import pallas as pl
from jax.experimental.pallas import tpu as pltpu
from jax.experimental.pallas import tpu_sc as plsc

H = 128
NC = 2
NS = 16
EB = 40
CMAX = 12544
DUMMY = CMAX


def _mlp_body(x_ref, w1_ref, b1_ref, w2_ref, b2_ref, w3_ref, b3_ref, o_ref):
    x = x_ref[...]
    h = jnp.maximum(jnp.dot(x, w1_ref[...], preferred_element_type=jnp.float32) + b1_ref[...], 0.0)
    h = jnp.maximum(jnp.dot(h, w2_ref[...], preferred_element_type=jnp.float32) + b2_ref[...], 0.0)
    o_ref[...] = jnp.dot(h, w3_ref[...], preferred_element_type=jnp.float32) + b3_ref[...]


def _mlp(x, p, name, blk=1000):
    n = x.shape[0]
    d = x.shape[1]
    w1 = p["mlp_%s_W1" % name]; b1 = p["mlp_%s_b1" % name].reshape(1, H)
    w2 = p["mlp_%s_W2" % name]; b2 = p["mlp_%s_b2" % name].reshape(1, H)
    w3 = p["mlp_%s_W3" % name]; b3 = p["mlp_%s_b3" % name].reshape(1, H)
    full = lambda r, c: pl.BlockSpec((r, c), lambda i: (0, 0))
    return pl.pallas_call(
        _mlp_body,
        grid=(n // blk,),
        in_specs=[
            pl.BlockSpec((blk, d), lambda i: (i, 0)),
            full(d, H), full(1, H), full(H, H), full(1, H), full(H, H), full(1, H),
        ],
        out_specs=pl.BlockSpec((blk, H), lambda i: (i, 0)),
        out_shape=jax.ShapeDtypeStruct((n, H), jnp.float32),
    )(x, w1, b1, w2, b2, w3, b3)


def _combine2_body(s1_ref, c1_ref, s2_ref, c2_ref, x_ref, wl1_ref, wl2_ref,
                   wr1_ref, wr2_ref, b_ref, o_ref):
    inv1 = 1.0 / jnp.maximum(c1_ref[:, 0:1], 1.0)
    inv2 = 1.0 / jnp.maximum(c2_ref[:, 0:1], 1.0)
    m1 = s1_ref[...] * inv1
    m2 = s2_ref[...] * inv2
    acc = jnp.dot(m1, wl1_ref[...], preferred_element_type=jnp.float32)
    acc += jnp.dot(m2, wl2_ref[...], preferred_element_type=jnp.float32)
    acc += jnp.dot(x_ref[...], wr1_ref[...] + wr2_ref[...], preferred_element_type=jnp.float32)
    o_ref[...] = jnp.maximum(acc + b_ref[...], 0.0)


def _combine2(s1, c1, s2, c2, x, wl1, wl2, wr1, wr2, b, blk=1000):
    n = x.shape[0]
    full = lambda r, c: pl.BlockSpec((r, c), lambda i: (0, 0))
    row = lambda c: pl.BlockSpec((blk, c), lambda i: (i, 0))
    return pl.pallas_call(
        _combine2_body,
        grid=(n // blk,),
        in_specs=[row(H), row(H), row(H), row(H), row(H),
                  full(H, H), full(H, H), full(H, H), full(H, H), full(1, H)],
        out_specs=row(H),
        out_shape=jax.ShapeDtypeStruct((n, H), jnp.float32),
    )(s1, c1, s2, c2, x, wl1, wl2, wr1, wr2, b)


def _combine1_body(s_ref, c_ref, x_ref, wl_ref, wr_ref, b_ref, o_ref):
    inv = 1.0 / jnp.maximum(c_ref[:, 0:1], 1.0)
    acc = jnp.dot(s_ref[...] * inv, wl_ref[...], preferred_element_type=jnp.float32)
    acc += jnp.dot(x_ref[...], wr_ref[...], preferred_element_type=jnp.float32)
    o_ref[...] = jnp.maximum(acc + b_ref[...], 0.0)


def _combine1(s, c, x, wl, wr, b, blk=1000):
    n = x.shape[0]
    full = lambda r, cc: pl.BlockSpec((r, cc), lambda i: (0, 0))
    row = lambda cc: pl.BlockSpec((blk, cc), lambda i: (i, 0))
    return pl.pallas_call(
        _combine1_body,
        grid=(n // blk,),
        in_specs=[row(H), row(H), row(H), full(H, H), full(H, H), full(1, H)],
        out_specs=row(H),
        out_shape=jax.ShapeDtypeStruct((n, H), jnp.float32),
    )(s, c, x, wl, wr, b)


def _final_body(s_ref, c_ref, x_ref, wl_ref, wr_ref, b_ref, lw_ref, lb_ref,
                bs_ref, o_ref, *, blk):
    inv = 1.0 / jnp.maximum(c_ref[:, 0:1], 1.0)
    acc = jnp.dot(s_ref[...] * inv, wl_ref[...], preferred_element_type=jnp.float32)
    acc += jnp.dot(x_ref[...], wr_ref[...], preferred_element_type=jnp.float32)
    h = jnp.maximum(acc + b_ref[...], 0.0)
    o = jnp.maximum(jnp.dot(h, lw_ref[...], preferred_element_type=jnp.float32) + lb_ref[...], 0.0)
    rows = pl.program_id(0) * blk + lax.broadcasted_iota(jnp.int32, (blk, 1), 0)
    o_ref[...] = jnp.where(rows < bs_ref[0, 0], o, 0.0)


def _final(s, c, x, wl, wr, b, lw, lb, bs, blk=1000):
    n = x.shape[0]
    full = lambda r, cc: pl.BlockSpec((r, cc), lambda i: (0, 0))
    row = lambda cc: pl.BlockSpec((blk, cc), lambda i: (i, 0))
    return pl.pallas_call(
        functools.partial(_final_body, blk=blk),
        grid=(n // blk,),
        in_specs=[row(H), row(H), row(H), full(H, H), full(H, H), full(1, H),
                  full(H, 1), full(1, 1),
                  pl.BlockSpec(memory_space=pltpu.SMEM)],
        out_specs=row(1),
        out_shape=jax.ShapeDtypeStruct((n, 1), jnp.float32),
    )(s, c, x, wl, wr, b, lw, lb, bs)


def _epad(E):
    m = 5120
    return ((E + m - 1) // m) * m


def _sc_chunk_lidx(cols, E, chunk, n_chunks):
    ep = _epad(E)
    er = ep // 128
    br = next(b for b in (512, 320, 256, 160, 128, 64, 32, 16, 8) if er % b == 0)
    cols_p = jnp.pad(cols, (0, ep - E), constant_values=2 ** 30).reshape(er, 128)

    def body(c_ref, o_ref):
        ch = pl.program_id(1)
        lo = ch * chunk
        v = c_ref[...]
        m = (v >= lo) & (v < lo + chunk)
        o_ref[...] = jnp.where(m, v - lo, DUMMY)

    out = pl.pallas_call(
        body,
        grid=(er // br, n_chunks),
        in_specs=[pl.BlockSpec((br, 128), lambda i, ch: (i, 0))],
        out_specs=pl.BlockSpec((br, 128),
                               lambda i, ch, _g=er // br: (ch * _g + i, 0)),
        out_shape=jax.ShapeDtypeStruct((n_chunks * er, 128), jnp.int32),
    )(cols_p)
    return out.reshape(n_chunks * ep)


def _make_sc_segsum(task_shapes):
    out_type = []
    for (E, n_dst, chunk, with_count) in task_shapes:
        assert n_dst % chunk == 0 and chunk % (16 * NS) == 0
        assert (n_dst // chunk) % NC == 0 and chunk <= CMAX
        out_type.append(jax.ShapeDtypeStruct((n_dst, H), jnp.float32))
        if with_count:
            out_type.append(jax.ShapeDtypeStruct((n_dst, H), jnp.float32))

    mesh = plsc.VectorSubcoreMesh(core_axis_name="c", subcore_axis_name="s")
    NB = 4
    scratch_types = (
        [pltpu.VMEM_SHARED((CMAX + 8, H), jnp.float32)]
        + [pltpu.VMEM((8, H), jnp.float32)]
        + [pltpu.VMEM((EB,), jnp.int32) for _ in range(NB)]
        + [pltpu.VMEM((EB,), jnp.int32) for _ in range(NB)]
        + [pltpu.VMEM((EB, H), jnp.float32) for _ in range(NB)]
        + [pltpu.SemaphoreType.DMA for _ in range(2 * NB)]
    )

    @functools.partial(pl.kernel, mesh=mesh, out_type=tuple(out_type),
                       scratch_types=scratch_types)
    def sc_kernel(*refs):
        n_task = len(task_shapes)
        ins = refs[:3 * n_task + 1]
        ones_in = ins[3 * n_task]
        outs = list(refs[3 * n_task + 1:len(refs) - len(scratch_types)])
        sc = refs[len(refs) - len(scratch_types):]
        feat_acc, zfeat = sc[0], sc[1]
        rb = sc[2:2 + NB]
        sb = sc[2 + NB:2 + 2 * NB]
        gb = sc[2 + 2 * NB:2 + 3 * NB]
        gsem = sc[2 + 3 * NB:2 + 3 * NB + NB]
        ssem = sc[2 + 3 * NB + NB:]

        cid = lax.axis_index("c")
        sid = lax.axis_index("s")

        zero = jnp.zeros((16,), jnp.float32)
        for r in range(8):
            for k in range(H // 16):
                zfeat[r, pl.ds(16 * k, 16)] = zero

        def drain_scatter(b, src_hbm):
            pltpu.make_async_copy(src_hbm.at[pl.ds(0, EB)], gb[b],
                                  ssem[b]).wait()

        oi = 0
        for t, (E, n_dst, chunk, with_count) in enumerate(task_shapes):
            rows_hbm, lidx_hbm, src_hbm = ins[3 * t], ins[3 * t + 1], ins[3 * t + 2]
            s_hbm = outs[oi]; oi += 1
            c_hbm = outs[oi] if with_count else None
            if with_count:
                oi += 1

            ep = _epad(E)
            nsub = ep // EB // NS
            rps = chunk // NS
            my_base = sid * nsub * EB

            def zero_acc(_rps=rps):
                def zbody(i, _b=sid * _rps):
                    pltpu.sync_copy(zfeat, feat_acc.at[pl.ds(_b + i * 8, 8)])
                pl.loop(0, _rps // 8)(zbody)

            def copy_out(dst_hbm, lo, _rps=rps):
                def cpbody(i, _b=sid * _rps, _lo=lo, _d=dst_hbm):
                    r = _b + i * 16
                    pltpu.sync_copy(feat_acc.at[pl.ds(r, 16)],
                                    _d.at[pl.ds(_lo + r, 16)])
                pl.loop(0, _rps // 16)(cpbody)

            for p in range((n_dst // chunk) // NC):
                ch = p * NC + cid
                lo = ch * chunk
                lbase = ch * ep + my_base

                zero_acc()
                plsc.subcore_barrier()

                for b in range(2):
                    off = b * EB
                    pltpu.sync_copy(rows_hbm.at[pl.ds(my_base + off, EB)], rb[b])
                    pltpu.sync_copy(lidx_hbm.at[pl.ds(lbase + off, EB)], sb[b])
                    pltpu.async_copy(src_hbm.at[rb[b]], gb[b], gsem[b])

                def fbody(i4, _lb=lbase, _n=nsub):
                    for b in range(NB):
                        i = i4 * NB + b
                        pltpu.make_async_copy(src_hbm.at[pl.ds(0, EB)], gb[b],
                                              gsem[b]).wait()
                        pltpu.async_copy(gb[b], feat_acc.at[sb[b]], ssem[b],
                                         add=True)
                        @pl.when(i < _n - 2)
                        def _(b=b, i=i):
                            t = (b + 2) % NB
                            @pl.when(i >= 2)
                            def _():
                                drain_scatter(t, src_hbm)
                            off = (i + 2) * EB
                            pltpu.sync_copy(rows_hbm.at[pl.ds(my_base + off, EB)],
                                            rb[t])
                            pltpu.sync_copy(lidx_hbm.at[pl.ds(_lb + off, EB)],
                                            sb[t])
                            pltpu.async_copy(src_hbm.at[rb[t]], gb[t], gsem[t])
                pl.loop(0, nsub // NB)(fbody)
                for b in range(NB):
                    drain_scatter(b, src_hbm)
                plsc.subcore_barrier()
                copy_out(s_hbm, lo)

                if with_count:
                    plsc.subcore_barrier()
                    zero_acc()
                    pltpu.sync_copy(ones_in, gb[0])
                    plsc.subcore_barrier()
                    for b in range(2):
                        pltpu.sync_copy(lidx_hbm.at[pl.ds(lbase + b * EB, EB)],
                                        sb[b])
                        pltpu.async_copy(gb[0], feat_acc.at[sb[b]], ssem[b],
                                         add=True)

                    def cbody(i4, _lb=lbase, _n=nsub):
                        for b in range(NB):
                            i = i4 * NB + b
                            @pl.when(i < _n - 2)
                            def _(b=b, i=i):
                                t = (b + 2) % NB
                                @pl.when(i >= 2)
                                def _():
                                    drain_scatter(t, src_hbm)
                                pltpu.sync_copy(
                                    lidx_hbm.at[pl.ds(_lb + (i + 2) * EB, EB)],
                                    sb[t])
                                pltpu.async_copy(gb[0], feat_acc.at[sb[t]],
                                                 ssem[t], add=True)
                    pl.loop(0, nsub // NB)(cbody)
                    for b in range(NB):
                        drain_scatter(b, src_hbm)
                    plsc.subcore_barrier()
                    copy_out(c_hbm, lo)

    return sc_kernel


N_SNP_PAD = 50176
N_GENE_PAD = 20480


def _pad_rows(ei):
    E = ei.shape[1]
    return jnp.pad(ei[0], (0, _epad(E) - E))


def _sc_layer1(h_snp, h_gene, ei_sg, ei_gg, ei_gs, lidx_gs):
    lidx_sg = _sc_chunk_lidx(ei_sg[1], 200000, 10240, 2)
    lidx_gg = _sc_chunk_lidx(ei_gg[1], 100000, 10240, 2)
    ones = jnp.ones((EB, H), jnp.float32)
    k = _make_sc_segsum([
        (200000, N_GENE_PAD, 10240, True),
        (100000, N_GENE_PAD, 10240, True),
        (200000, N_SNP_PAD, CMAX, True),
    ])
    return k(
        _pad_rows(ei_sg), lidx_sg, h_snp,
        _pad_rows(ei_gg), lidx_gg, h_gene,
        _pad_rows(ei_gs), lidx_gs, h_gene, ones)


def _sc_layer2(gene1, ei_gs, lidx_gs):
    ones = jnp.ones((EB, H), jnp.float32)
    k = _make_sc_segsum([
        (200000, N_SNP_PAD, CMAX, False),
    ])
    return k(_pad_rows(ei_gs), lidx_gs, gene1, ones)


def kernel(x_SNP, x_Gene, x_CC, x_BP, x_MF, ei_snp_gene, ei_gene_snp,
           ei_gene_gene, ei_gene_cc, ei_gene_bp, ei_gene_mf, params, batch_size):
    p = params
    n_snp = x_SNP.shape[0]
    n_gene = x_Gene.shape[0]

    h_snp = _mlp(x_SNP, p, "snp")
    h_gene = _mlp(x_Gene, p, "gene")

    lidx_gs = _sc_chunk_lidx(ei_gene_snp[1], 200000, CMAX, 4)

    s_sg, c_sg, s_gg, c_gg, s_gs_p, c_gs_p = _sc_layer1(
        h_snp, h_gene, ei_snp_gene, ei_gene_gene, ei_gene_snp, lidx_gs)
    s_sg, c_sg = s_sg[:n_gene], c_sg[:n_gene]
    s_gg, c_gg = s_gg[:n_gene], c_gg[:n_gene]
    s_gs = s_gs_p[:n_snp]
    c_gs = c_gs_p[:n_snp]

    gene1 = _combine2(
        s_sg, c_sg, s_gg, c_gg, h_gene,
        p["conv0_snp_gene_Wl"], p["conv0_gene_gene_Wl"],
        p["conv0_snp_gene_Wr"], p["conv0_gene_gene_Wr"],
        (p["conv0_snp_gene_bl"] + p["conv0_gene_gene_bl"]).reshape(1, H))
    snp1 = _combine1(
        s_gs, c_gs, h_snp,
        p["conv0_gene_snp_Wl"], p["conv0_gene_snp_Wr"],
        p["conv0_gene_snp_bl"].reshape(1, H))

    s2 = _sc_layer2(gene1, ei_gene_snp, lidx_gs)
    s2_gs = (s2[0] if isinstance(s2, (tuple, list)) else s2)[:n_snp]

    bs = jnp.asarray(batch_size, jnp.int32).reshape(1, 1)
    out = _final(
        s2_gs, c_gs, snp1,
        p["conv1_gene_snp_Wl"], p["conv1_gene_snp_Wr"],
        p["conv1_gene_snp_bl"].reshape(1, H),
        p["lin_W"], p["lin_b"].reshape(1, 1), bs)
    return out


# speedup vs baseline: 1.1614x; 1.0196x over previous
import functools

import jax
import jax.numpy as jnp
from jax import lax
from jax.experimental import pallas as pl
from jax.experimental.pallas import tpu as pltpu
from jax.experimental.pallas import tpu_sc as plsc

H = 128
NC = 2
NS = 16
EB = 40
CMAX = 12544
DUMMY = CMAX


def _mlp_body(x_ref, w1_ref, b1_ref, w2_ref, b2_ref, w3_ref, b3_ref, o_ref):
    x = x_ref[...]
    h = jnp.maximum(jnp.dot(x, w1_ref[...], preferred_element_type=jnp.float32) + b1_ref[...], 0.0)
    h = jnp.maximum(jnp.dot(h, w2_ref[...], preferred_element_type=jnp.float32) + b2_ref[...], 0.0)
    o_ref[...] = jnp.dot(h, w3_ref[...], preferred_element_type=jnp.float32) + b3_ref[...]


def _mlp(x, p, name, blk=1000):
    n = x.shape[0]
    d = x.shape[1]
    w1 = p["mlp_%s_W1" % name]; b1 = p["mlp_%s_b1" % name].reshape(1, H)
    w2 = p["mlp_%s_W2" % name]; b2 = p["mlp_%s_b2" % name].reshape(1, H)
    w3 = p["mlp_%s_W3" % name]; b3 = p["mlp_%s_b3" % name].reshape(1, H)
    full = lambda r, c: pl.BlockSpec((r, c), lambda i: (0, 0))
    return pl.pallas_call(
        _mlp_body,
        grid=(n // blk,),
        in_specs=[
            pl.BlockSpec((blk, d), lambda i: (i, 0)),
            full(d, H), full(1, H), full(H, H), full(1, H), full(H, H), full(1, H),
        ],
        out_specs=pl.BlockSpec((blk, H), lambda i: (i, 0)),
        out_shape=jax.ShapeDtypeStruct((n, H), jnp.float32),
    )(x, w1, b1, w2, b2, w3, b3)


def _combine2_body(s1_ref, c1_ref, s2_ref, c2_ref, x_ref, wl1_ref, wl2_ref,
                   wr1_ref, wr2_ref, b_ref, o_ref):
    inv1 = 1.0 / jnp.maximum(c1_ref[:, 0:1], 1.0)
    inv2 = 1.0 / jnp.maximum(c2_ref[:, 0:1], 1.0)
    m1 = s1_ref[...] * inv1
    m2 = s2_ref[...] * inv2
    acc = jnp.dot(m1, wl1_ref[...], preferred_element_type=jnp.float32)
    acc += jnp.dot(m2, wl2_ref[...], preferred_element_type=jnp.float32)
    acc += jnp.dot(x_ref[...], wr1_ref[...] + wr2_ref[...], preferred_element_type=jnp.float32)
    o_ref[...] = jnp.maximum(acc + b_ref[...], 0.0)


def _combine2(s1, c1, s2, c2, x, wl1, wl2, wr1, wr2, b, blk=1000):
    n = x.shape[0]
    full = lambda r, c: pl.BlockSpec((r, c), lambda i: (0, 0))
    row = lambda c: pl.BlockSpec((blk, c), lambda i: (i, 0))
    return pl.pallas_call(
        _combine2_body,
        grid=(n // blk,),
        in_specs=[row(H), row(H), row(H), row(H), row(H),
                  full(H, H), full(H, H), full(H, H), full(H, H), full(1, H)],
        out_specs=row(H),
        out_shape=jax.ShapeDtypeStruct((n, H), jnp.float32),
    )(s1, c1, s2, c2, x, wl1, wl2, wr1, wr2, b)


def _combine1_body(s_ref, c_ref, x_ref, wl_ref, wr_ref, b_ref, o_ref):
    inv = 1.0 / jnp.maximum(c_ref[:, 0:1], 1.0)
    acc = jnp.dot(s_ref[...] * inv, wl_ref[...], preferred_element_type=jnp.float32)
    acc += jnp.dot(x_ref[...], wr_ref[...], preferred_element_type=jnp.float32)
    o_ref[...] = jnp.maximum(acc + b_ref[...], 0.0)


def _combine1(s, c, x, wl, wr, b, blk=1000):
    n = x.shape[0]
    full = lambda r, cc: pl.BlockSpec((r, cc), lambda i: (0, 0))
    row = lambda cc: pl.BlockSpec((blk, cc), lambda i: (i, 0))
    return pl.pallas_call(
        _combine1_body,
        grid=(n // blk,),
        in_specs=[row(H), row(H), row(H), full(H, H), full(H, H), full(1, H)],
        out_specs=row(H),
        out_shape=jax.ShapeDtypeStruct((n, H), jnp.float32),
    )(s, c, x, wl, wr, b)


def _final_body(s_ref, c_ref, x_ref, wl_ref, wr_ref, b_ref, lw_ref, lb_ref,
                bs_ref, o_ref, *, blk):
    inv = 1.0 / jnp.maximum(c_ref[:, 0:1], 1.0)
    acc = jnp.dot(s_ref[...] * inv, wl_ref[...], preferred_element_type=jnp.float32)
    acc += jnp.dot(x_ref[...], wr_ref[...], preferred_element_type=jnp.float32)
    h = jnp.maximum(acc + b_ref[...], 0.0)
    o = jnp.maximum(jnp.dot(h, lw_ref[...], preferred_element_type=jnp.float32) + lb_ref[...], 0.0)
    rows = pl.program_id(0) * blk + lax.broadcasted_iota(jnp.int32, (blk, 1), 0)
    o_ref[...] = jnp.where(rows < bs_ref[0, 0], o, 0.0)


def _final(s, c, x, wl, wr, b, lw, lb, bs, blk=1000):
    n = x.shape[0]
    full = lambda r, cc: pl.BlockSpec((r, cc), lambda i: (0, 0))
    row = lambda cc: pl.BlockSpec((blk, cc), lambda i: (i, 0))
    return pl.pallas_call(
        functools.partial(_final_body, blk=blk),
        grid=(n // blk,),
        in_specs=[row(H), row(H), row(H), full(H, H), full(H, H), full(1, H),
                  full(H, 1), full(1, 1),
                  pl.BlockSpec(memory_space=pltpu.SMEM)],
        out_specs=row(1),
        out_shape=jax.ShapeDtypeStruct((n, 1), jnp.float32),
    )(s, c, x, wl, wr, b, lw, lb, bs)


def _epad(E):
    m = 5120
    return ((E + m - 1) // m) * m


def _sc_chunk_lidx(cols, E, chunk, n_chunks):
    ep = _epad(E)
    er = ep // 128
    br = next(b for b in (512, 320, 256, 160, 128, 64, 32, 16, 8) if er % b == 0)
    cols_p = jnp.pad(cols, (0, ep - E), constant_values=2 ** 30).reshape(er, 128)

    def body(c_ref, o_ref):
        ch = pl.program_id(1)
        lo = ch * chunk
        v = c_ref[...]
        m = (v >= lo) & (v < lo + chunk)
        o_ref[...] = jnp.where(m, v - lo, DUMMY)

    out = pl.pallas_call(
        body,
        grid=(er // br, n_chunks),
        in_specs=[pl.BlockSpec((br, 128), lambda i, ch: (i, 0))],
        out_specs=pl.BlockSpec((br, 128),
                               lambda i, ch, _g=er // br: (ch * _g + i, 0)),
        out_shape=jax.ShapeDtypeStruct((n_chunks * er, 128), jnp.int32),
    )(cols_p)
    return out.reshape(n_chunks * ep // EB, EB)


def _make_sc_segsum(task_shapes):
    out_type = []
    for (E, n_dst, chunk, with_count) in task_shapes:
        assert n_dst % chunk == 0 and chunk % (16 * NS) == 0
        assert (n_dst // chunk) % NC == 0 and chunk <= CMAX
        out_type.append(jax.ShapeDtypeStruct((n_dst, H), jnp.float32))
        if with_count:
            out_type.append(jax.ShapeDtypeStruct((n_dst, H), jnp.float32))

    mesh = plsc.VectorSubcoreMesh(core_axis_name="c", subcore_axis_name="s")
    NB = 4
    GRP = 8
    scratch_types = (
        [pltpu.VMEM_SHARED((CMAX + 8, H), jnp.float32)]
        + [pltpu.VMEM((8, H), jnp.float32)]
        + [pltpu.VMEM((GRP, EB), jnp.int32)]
        + [pltpu.VMEM((GRP, EB), jnp.int32)]
        + [pltpu.VMEM((EB, H), jnp.float32) for _ in range(NB)]
        + [pltpu.SemaphoreType.DMA for _ in range(2 * NB)]
    )

    @functools.partial(pl.kernel, mesh=mesh, out_type=tuple(out_type),
                       scratch_types=scratch_types)
    def sc_kernel(*refs):
        n_task = len(task_shapes)
        ins = refs[:3 * n_task + 1]
        ones_in = ins[3 * n_task]
        outs = list(refs[3 * n_task + 1:len(refs) - len(scratch_types)])
        sc = refs[len(refs) - len(scratch_types):]
        feat_acc, zfeat, rbig, sbig = sc[0], sc[1], sc[2], sc[3]
        gb = sc[4:4 + NB]
        gsem = sc[4 + NB:4 + 2 * NB]
        ssem = sc[4 + 2 * NB:]

        cid = lax.axis_index("c")
        sid = lax.axis_index("s")

        zero = jnp.zeros((16,), jnp.float32)
        for r in range(8):
            for k in range(H // 16):
                zfeat[r, pl.ds(16 * k, 16)] = zero

        def drain_scatter(b, src_hbm):
            pltpu.make_async_copy(src_hbm.at[pl.ds(0, EB)], gb[b],
                                  ssem[b]).wait()

        oi = 0
        for t, (E, n_dst, chunk, with_count) in enumerate(task_shapes):
            rows_hbm, lidx_hbm, src_hbm = ins[3 * t], ins[3 * t + 1], ins[3 * t + 2]
            s_hbm = outs[oi]; oi += 1
            c_hbm = outs[oi] if with_count else None
            if with_count:
                oi += 1

            ep = _epad(E)
            nsub = ep // EB // NS
            ngrp = nsub // GRP
            nbt = ep // EB
            rps = chunk // NS
            my_row = sid * nsub

            def zero_acc(_rps=rps):
                def zbody(i, _b=sid * _rps):
                    pltpu.sync_copy(zfeat, feat_acc.at[pl.ds(_b + i * 8, 8)])
                pl.loop(0, _rps // 8)(zbody)

            def copy_out(dst_hbm, lo, _rps=rps):
                def cpbody(i, _b=sid * _rps, _lo=lo, _d=dst_hbm):
                    r = _b + i * 16
                    pltpu.sync_copy(feat_acc.at[pl.ds(r, 16)],
                                    _d.at[pl.ds(_lo + r, 16)])
                pl.loop(0, _rps // 16)(cpbody)

            for p in range((n_dst // chunk) // NC):
                ch = p * NC + cid
                lo = ch * chunk
                lrow = ch * nbt + my_row

                zero_acc()
                plsc.subcore_barrier()

                def fbody(g, _lrow=lrow):
                    gr = g * GRP
                    pltpu.sync_copy(rows_hbm.at[pl.ds(my_row + gr, GRP)], rbig)
                    pltpu.sync_copy(lidx_hbm.at[pl.ds(_lrow + gr, GRP)], sbig)
                    for k in range(2):
                        pltpu.async_copy(src_hbm.at[rbig.at[k]], gb[k], gsem[k])
                    for k in range(GRP):
                        s = k % NB
                        pltpu.make_async_copy(src_hbm.at[pl.ds(0, EB)], gb[s],
                                              gsem[s]).wait()
                        pltpu.async_copy(gb[s], feat_acc.at[sbig.at[k]],
                                         ssem[s], add=True)
                        if k + 2 < GRP:
                            t = (k + 2) % NB
                            if k >= 2:
                                drain_scatter(t, src_hbm)
                            pltpu.async_copy(src_hbm.at[rbig.at[k + 2]], gb[t],
                                             gsem[t])
                    for k in range(GRP - 2, GRP):
                        drain_scatter(k % NB, src_hbm)
                pl.loop(0, ngrp)(fbody)
                plsc.subcore_barrier()
                copy_out(s_hbm, lo)

                if with_count:
                    plsc.subcore_barrier()
                    zero_acc()
                    pltpu.sync_copy(ones_in, gb[0])
                    plsc.subcore_barrier()

                    def cbody(g, _lrow=lrow):
                        gr = g * GRP
                        pltpu.sync_copy(lidx_hbm.at[pl.ds(_lrow + gr, GRP)],
                                        sbig)
                        for k in range(GRP):
                            s = k % NB
                            if k >= NB:
                                drain_scatter(s, src_hbm)
                            pltpu.async_copy(gb[0], feat_acc.at[sbig.at[k]],
                                             ssem[s], add=True)
                        for s in range(NB):
                            drain_scatter(s, src_hbm)
                    pl.loop(0, ngrp)(cbody)
                    plsc.subcore_barrier()
                    copy_out(c_hbm, lo)

    return sc_kernel


N_SNP_PAD = 50176
N_GENE_PAD = 20480


def _pad_rows(ei):
    E = ei.shape[1]
    return jnp.pad(ei[0], (0, _epad(E) - E)).reshape(-1, EB)


def _sc_layer1(h_snp, h_gene, ei_sg, ei_gg, ei_gs, lidx_gs):
    lidx_sg = _sc_chunk_lidx(ei_sg[1], 200000, 10240, 2)
    lidx_gg = _sc_chunk_lidx(ei_gg[1], 100000, 10240, 2)
    ones = jnp.ones((EB, H), jnp.float32)
    k = _make_sc_segsum([
        (200000, N_GENE_PAD, 10240, True),
        (100000, N_GENE_PAD, 10240, True),
        (200000, N_SNP_PAD, CMAX, True),
    ])
    return k(
        _pad_rows(ei_sg), lidx_sg, h_snp,
        _pad_rows(ei_gg), lidx_gg, h_gene,
        _pad_rows(ei_gs), lidx_gs, h_gene, ones)


def _sc_layer2(gene1, ei_gs, lidx_gs):
    ones = jnp.ones((EB, H), jnp.float32)
    k = _make_sc_segsum([
        (200000, N_SNP_PAD, CMAX, False),
    ])
    return k(_pad_rows(ei_gs), lidx_gs, gene1, ones)


def kernel(x_SNP, x_Gene, x_CC, x_BP, x_MF, ei_snp_gene, ei_gene_snp,
           ei_gene_gene, ei_gene_cc, ei_gene_bp, ei_gene_mf, params, batch_size):
    p = params
    n_snp = x_SNP.shape[0]
    n_gene = x_Gene.shape[0]

    h_snp = _mlp(x_SNP, p, "snp")
    h_gene = _mlp(x_Gene, p, "gene")

    lidx_gs = _sc_chunk_lidx(ei_gene_snp[1], 200000, CMAX, 4)

    s_sg, c_sg, s_gg, c_gg, s_gs_p, c_gs_p = _sc_layer1(
        h_snp, h_gene, ei_snp_gene, ei_gene_gene, ei_gene_snp, lidx_gs)
    s_sg, c_sg = s_sg[:n_gene], c_sg[:n_gene]
    s_gg, c_gg = s_gg[:n_gene], c_gg[:n_gene]
    s_gs = s_gs_p[:n_snp]
    c_gs = c_gs_p[:n_snp]

    gene1 = _combine2(
        s_sg, c_sg, s_gg, c_gg, h_gene,
        p["conv0_snp_gene_Wl"], p["conv0_gene_gene_Wl"],
        p["conv0_snp_gene_Wr"], p["conv0_gene_gene_Wr"],
        (p["conv0_snp_gene_bl"] + p["conv0_gene_gene_bl"]).reshape(1, H))
    snp1 = _combine1(
        s_gs, c_gs, h_snp,
        p["conv0_gene_snp_Wl"], p["conv0_gene_snp_Wr"],
        p["conv0_gene_snp_bl"].reshape(1, H))

    s2 = _sc_layer2(gene1, ei_gene_snp, lidx_gs)
    s2_gs = (s2[0] if isinstance(s2, (tuple, list)) else s2)[:n_snp]

    bs = jnp.asarray(batch_size, jnp.int32).reshape(1, 1)
    out = _final(
        s2_gs, c_gs, snp1,
        p["conv1_gene_snp_Wl"], p["conv1_gene_snp_Wr"],
        p["conv1_gene_snp_bl"].reshape(1, H),
        p["lin_W"], p["lin_b"].reshape(1, 1), bs)
    return out
